# baseline (speedup 1.0000x reference)
import functools

import jax
import jax.numpy as jnp
from jax import lax
from jax.experimental import pallas as pl
from jax.experimental.pallas import tpu as pltpu
from jax.experimental.pallas import tpu_sc as plsc

HIDDEN = 2048
NUM_EXPERTS = 16
TOP_K = 2

_NC = 2
_NS = 16
_NW = _NC * _NS
_RPW = HIDDEN // _NW
_CHUNK = 16
_NCHUNK = _RPW // _CHUNK


def _router_kernel(x_ref, W1_ref, b1_ref, W2_ref, b2_ref, eb_ref,
                   idx_ref, w_ref, bc_ref):
    pooled = jnp.mean(x_ref[0], axis=0, keepdims=True)
    h = jnp.dot(pooled, W1_ref[...], preferred_element_type=jnp.float32)
    h = h + b1_ref[...]
    h = h * jax.nn.sigmoid(h)
    logits = jnp.dot(h, W2_ref[...], preferred_element_type=jnp.float32)
    logits = logits + b2_ref[...]

    iota = jax.lax.broadcasted_iota(jnp.int32, (1, NUM_EXPERTS), 1)
    m0 = jnp.max(logits)
    i0 = jnp.min(jnp.where(logits == m0, iota, NUM_EXPERTS))
    masked = jnp.where(iota == i0, -jnp.inf, logits)
    m1 = jnp.max(masked)
    i1 = jnp.min(jnp.where(masked == m1, iota, NUM_EXPERTS))
    w0 = 1.0 / (1.0 + jnp.exp(m1 - m0))
    w1 = 1.0 - w0

    idx_ref[0] = i0
    idx_ref[1] = i1
    w_ref[0] = w0
    w_ref[1] = w1

    wvec = jnp.where(iota == i0, w0, 0.0) + jnp.where(iota == i1, w1, 0.0)
    bc_ref[...] = jnp.dot(wvec, eb_ref[...], preferred_element_type=jnp.float32)


def _sc_combine(ew_ref, ridx_ref, w0_ref, w1_ref, out_ref,
                idx_a, idx_b, wv0, wv1, a_buf, b_buf, c_buf, sem_a, sem_b):
    wid = lax.axis_index("s") * _NC + lax.axis_index("c")
    base = wid * _RPW

    pltpu.sync_copy(w0_ref, wv0)
    pltpu.sync_copy(w1_ref, wv1)

    for c in range(_NCHUNK):
        rowbase = base + c * _CHUNK
        pltpu.sync_copy(ridx_ref.at[pl.ds(rowbase, _CHUNK)], idx_a)
        pltpu.sync_copy(ridx_ref.at[pl.ds(HIDDEN + rowbase, _CHUNK)], idx_b)
        cp_a = pltpu.make_async_copy(ew_ref.at[idx_a], a_buf, sem_a)
        cp_a.start()
        cp_b = pltpu.make_async_copy(ew_ref.at[idx_b], b_buf, sem_b)
        cp_b.start()
        cp_a.wait()
        cp_b.wait()

        w0v = wv0[...]
        w1v = wv1[...]

        def body(j, carry):
            for r in range(_CHUNK):
                sl = pl.ds(j * 16, 16)
                c_buf[r, sl] = a_buf[r, sl] * w0v + b_buf[r, sl] * w1v
            return carry

        lax.fori_loop(0, HIDDEN // 16, body, 0)
        pltpu.sync_copy(c_buf, out_ref.at[pl.ds(rowbase, _CHUNK)])


_sc_combine_call = functools.partial(
    pl.kernel,
    mesh=plsc.VectorSubcoreMesh(core_axis_name="c", subcore_axis_name="s"),
    out_type=jax.ShapeDtypeStruct((HIDDEN, HIDDEN), jnp.float32),
    scratch_types=[
        pltpu.VMEM((_CHUNK,), jnp.int32),
        pltpu.VMEM((_CHUNK,), jnp.int32),
        pltpu.VMEM((16,), jnp.float32),
        pltpu.VMEM((16,), jnp.float32),
        pltpu.VMEM((_CHUNK, HIDDEN), jnp.float32),
        pltpu.VMEM((_CHUNK, HIDDEN), jnp.float32),
        pltpu.VMEM((_CHUNK, HIDDEN), jnp.float32),
        pltpu.SemaphoreType.DMA,
        pltpu.SemaphoreType.DMA,
    ],
)(_sc_combine)


def _gemm_kernel(x_ref, wc_ref, bc_ref, out_ref, wcb_ref):
    @pl.when(jnp.logical_and(pl.program_id(0) == 0, pl.program_id(1) == 0))
    def _pack():
        wcb_ref[...] = wc_ref[...].astype(jnp.bfloat16)

    acc = jnp.dot(x_ref[0].astype(jnp.bfloat16), wcb_ref[...],
                  preferred_element_type=jnp.float32)
    out_ref[0] = acc + bc_ref[...]


@jax.jit
def kernel(x, W1, b1, W2, b2, expert_W, expert_b):
    B, S, H = x.shape
    E = expert_W.shape[0]

    idx, w, bc = pl.pallas_call(
        _router_kernel,
        grid=(1,),
        in_specs=[
            pl.BlockSpec((1, S, H), lambda i: (0, 0, 0)),
            pl.BlockSpec((H, H // 2), lambda i: (0, 0)),
            pl.BlockSpec((1, H // 2), lambda i: (0, 0)),
            pl.BlockSpec((H // 2, NUM_EXPERTS), lambda i: (0, 0)),
            pl.BlockSpec((1, NUM_EXPERTS), lambda i: (0, 0)),
            pl.BlockSpec((NUM_EXPERTS, H), lambda i: (0, 0)),
        ],
        out_shape=[
            jax.ShapeDtypeStruct((TOP_K,), jnp.int32),
            jax.ShapeDtypeStruct((TOP_K,), jnp.float32),
            jax.ShapeDtypeStruct((1, H), jnp.float32),
        ],
        out_specs=[
            pl.BlockSpec(memory_space=pltpu.SMEM),
            pl.BlockSpec(memory_space=pltpu.SMEM),
            pl.BlockSpec((1, H), lambda i: (0, 0)),
        ],
    )(x, W1, b1.reshape(1, -1), W2, b2.reshape(1, -1), expert_b)

    ridx = (idx[:, None] * H
            + jnp.arange(H, dtype=jnp.int32)[None, :]).reshape(-1)
    w0_16 = jnp.full((16,), w[0], jnp.float32)
    w1_16 = jnp.full((16,), w[1], jnp.float32)
    wc = _sc_combine_call(expert_W.reshape(E * H, H), ridx, w0_16, w1_16)

    MTS = 512
    y = pl.pallas_call(
        _gemm_kernel,
        grid=(B, S // MTS),
        in_specs=[
            pl.BlockSpec((1, MTS, H), lambda b, s: (b, s, 0)),
            pl.BlockSpec((H, H), lambda b, s: (0, 0)),
            pl.BlockSpec((1, H), lambda b, s: (0, 0)),
        ],
        out_specs=pl.BlockSpec((1, MTS, H), lambda b, s: (b, s, 0)),
        out_shape=jax.ShapeDtypeStruct((B, S, H), jnp.float32),
        scratch_shapes=[pltpu.VMEM((H, H), jnp.bfloat16)],
    )(x, wc, bc)

    return y


# speedup vs baseline: 1.2164x; 1.2164x over previous
import functools

import jax
import jax.numpy as jnp
from jax.experimental import pallas as pl
from jax.experimental.pallas import tpu as pltpu

HIDDEN = 2048
NUM_EXPERTS = 16
TOP_K = 2


def _router_kernel(x_ref, W1_ref, b1_ref, W2_ref, b2_ref, eb_ref,
                   idx_ref, w_ref, bc_ref):
    pooled = jnp.mean(x_ref[0], axis=0, keepdims=True)
    h = jnp.dot(pooled, W1_ref[...], preferred_element_type=jnp.float32)
    h = h + b1_ref[...]
    h = h * jax.nn.sigmoid(h)
    logits = jnp.dot(h, W2_ref[...], preferred_element_type=jnp.float32)
    logits = logits + b2_ref[...]

    iota = jax.lax.broadcasted_iota(jnp.int32, (1, NUM_EXPERTS), 1)
    m0 = jnp.max(logits)
    i0 = jnp.min(jnp.where(logits == m0, iota, NUM_EXPERTS))
    masked = jnp.where(iota == i0, -jnp.inf, logits)
    m1 = jnp.max(masked)
    i1 = jnp.min(jnp.where(masked == m1, iota, NUM_EXPERTS))
    w0 = 1.0 / (1.0 + jnp.exp(m1 - m0))
    w1 = 1.0 - w0

    idx_ref[0] = i0
    idx_ref[1] = i1
    w_ref[0] = w0
    w_ref[1] = w1

    wvec = jnp.where(iota == i0, w0, 0.0) + jnp.where(iota == i1, w1, 0.0)
    bc_ref[...] = jnp.dot(wvec, eb_ref[...], preferred_element_type=jnp.float32)


def _moe_gemm_kernel(idx_ref, w_ref, xlo_ref, xhi_ref, w0_ref, w1_ref,
                     bc_ref, out_ref, wc_ref):
    @pl.when(jnp.logical_and(pl.program_id(0) == 0, pl.program_id(1) == 0))
    def _combine():
        wc = w_ref[0] * w0_ref[0] + w_ref[1] * w1_ref[0]
        wc_ref[...] = wc.astype(jnp.bfloat16)

    m2 = xlo_ref.shape[1]
    acc_lo = jnp.dot(xlo_ref[0].astype(jnp.bfloat16), wc_ref[...],
                     preferred_element_type=jnp.float32)
    acc_hi = jnp.dot(xhi_ref[0].astype(jnp.bfloat16), wc_ref[...],
                     preferred_element_type=jnp.float32)
    out_ref[0, :m2] = acc_lo + bc_ref[...]
    out_ref[0, m2:] = acc_hi + bc_ref[...]


@jax.jit
def kernel(x, W1, b1, W2, b2, expert_W, expert_b):
    B, S, H = x.shape

    idx, w, bc = pl.pallas_call(
        _router_kernel,
        grid=(1,),
        in_specs=[
            pl.BlockSpec((1, S, H), lambda i: (0, 0, 0)),
            pl.BlockSpec((H, H // 2), lambda i: (0, 0)),
            pl.BlockSpec((1, H // 2), lambda i: (0, 0)),
            pl.BlockSpec((H // 2, NUM_EXPERTS), lambda i: (0, 0)),
            pl.BlockSpec((1, NUM_EXPERTS), lambda i: (0, 0)),
            pl.BlockSpec((NUM_EXPERTS, H), lambda i: (0, 0)),
        ],
        out_shape=[
            jax.ShapeDtypeStruct((TOP_K,), jnp.int32),
            jax.ShapeDtypeStruct((TOP_K,), jnp.float32),
            jax.ShapeDtypeStruct((1, H), jnp.float32),
        ],
        out_specs=[
            pl.BlockSpec(memory_space=pltpu.SMEM),
            pl.BlockSpec(memory_space=pltpu.SMEM),
            pl.BlockSpec((1, H), lambda i: (0, 0)),
        ],
    )(x, W1, b1.reshape(1, -1), W2, b2.reshape(1, -1), expert_b)

    MTS = 256
    y = pl.pallas_call(
        _moe_gemm_kernel,
        grid_spec=pltpu.PrefetchScalarGridSpec(
            num_scalar_prefetch=2,
            grid=(B, S // MTS),
            in_specs=[
                pl.BlockSpec((1, MTS // 2, H), lambda b, s, idx, w: (b, 2 * s, 0)),
                pl.BlockSpec((1, MTS // 2, H), lambda b, s, idx, w: (b, 2 * s + 1, 0)),
                pl.BlockSpec((1, H, H), lambda b, s, idx, w: (idx[0], 0, 0)),
                pl.BlockSpec((1, H, H), lambda b, s, idx, w: (idx[1], 0, 0)),
                pl.BlockSpec((1, H), lambda b, s, idx, w: (0, 0)),
            ],
            out_specs=pl.BlockSpec((1, MTS, H), lambda b, s, idx, w: (b, s, 0)),
            scratch_shapes=[pltpu.VMEM((H, H), jnp.bfloat16)],
        ),
        out_shape=jax.ShapeDtypeStruct((B, S, H), jnp.float32),
        compiler_params=pltpu.CompilerParams(
            vmem_limit_bytes=100 * 1024 * 1024,
        ),
    )(idx, w, x, x, expert_W, expert_W, bc)

    return y


# speedup vs baseline: 1.3307x; 1.0940x over previous
import functools

import jax
import jax.numpy as jnp
from jax.experimental import pallas as pl
from jax.experimental.pallas import tpu as pltpu

HIDDEN = 2048
NUM_EXPERTS = 16
TOP_K = 2


def _router_kernel(x_ref, W1_ref, b1_ref, W2_ref, b2_ref, eb_ref,
                   idx_ref, w_ref, bc_ref):
    pooled = jnp.mean(x_ref[0], axis=0, keepdims=True)
    h = jnp.dot(pooled, W1_ref[...], preferred_element_type=jnp.float32)
    h = h + b1_ref[...]
    h = h * jax.nn.sigmoid(h)
    logits = jnp.dot(h, W2_ref[...], preferred_element_type=jnp.float32)
    logits = logits + b2_ref[...]

    iota = jax.lax.broadcasted_iota(jnp.int32, (1, NUM_EXPERTS), 1)
    m0 = jnp.max(logits)
    i0 = jnp.min(jnp.where(logits == m0, iota, NUM_EXPERTS))
    masked = jnp.where(iota == i0, -jnp.inf, logits)
    m1 = jnp.max(masked)
    i1 = jnp.min(jnp.where(masked == m1, iota, NUM_EXPERTS))
    w0 = 1.0 / (1.0 + jnp.exp(m1 - m0))
    w1 = 1.0 - w0

    idx_ref[0] = i0
    idx_ref[1] = i1
    w_ref[0] = w0
    w_ref[1] = w1

    wvec = jnp.where(iota == i0, w0, 0.0) + jnp.where(iota == i1, w1, 0.0)
    bc_ref[...] = jnp.dot(wvec, eb_ref[...], preferred_element_type=jnp.float32)


def _moe_gemm_kernel(idx_ref, w_ref, x_ref, w0_ref, w1_ref, bc_ref,
                     out_ref, wc_ref):
    @pl.when(jnp.logical_and(pl.program_id(0) == 0, pl.program_id(1) == 0))
    def _combine():
        wc = w_ref[0] * w0_ref[0] + w_ref[1] * w1_ref[0]
        wc_ref[...] = wc.astype(jnp.bfloat16)

    acc = jnp.dot(x_ref[0].astype(jnp.bfloat16), wc_ref[...],
                  preferred_element_type=jnp.float32)
    out_ref[0] = acc + bc_ref[...]


@jax.jit
def kernel(x, W1, b1, W2, b2, expert_W, expert_b):
    B, S, H = x.shape

    idx, w, bc = pl.pallas_call(
        _router_kernel,
        grid=(1,),
        in_specs=[
            pl.BlockSpec((1, S, H), lambda i: (0, 0, 0)),
            pl.BlockSpec((H, H // 2), lambda i: (0, 0)),
            pl.BlockSpec((1, H // 2), lambda i: (0, 0)),
            pl.BlockSpec((H // 2, NUM_EXPERTS), lambda i: (0, 0)),
            pl.BlockSpec((1, NUM_EXPERTS), lambda i: (0, 0)),
            pl.BlockSpec((NUM_EXPERTS, H), lambda i: (0, 0)),
        ],
        out_shape=[
            jax.ShapeDtypeStruct((TOP_K,), jnp.int32),
            jax.ShapeDtypeStruct((TOP_K,), jnp.float32),
            jax.ShapeDtypeStruct((1, H), jnp.float32),
        ],
        out_specs=[
            pl.BlockSpec(memory_space=pltpu.SMEM),
            pl.BlockSpec(memory_space=pltpu.SMEM),
            pl.BlockSpec((1, H), lambda i: (0, 0)),
        ],
    )(x, W1, b1.reshape(1, -1), W2, b2.reshape(1, -1), expert_b)

    MTS = 512
    y = pl.pallas_call(
        _moe_gemm_kernel,
        grid_spec=pltpu.PrefetchScalarGridSpec(
            num_scalar_prefetch=2,
            grid=(B, S // MTS),
            in_specs=[
                pl.BlockSpec((1, MTS, H), lambda b, s, idx, w: (b, s, 0)),
                pl.BlockSpec((1, H, H), lambda b, s, idx, w: (idx[0], 0, 0)),
                pl.BlockSpec((1, H, H), lambda b, s, idx, w: (idx[1], 0, 0)),
                pl.BlockSpec((1, H), lambda b, s, idx, w: (0, 0)),
            ],
            out_specs=pl.BlockSpec((1, MTS, H), lambda b, s, idx, w: (b, s, 0)),
            scratch_shapes=[pltpu.VMEM((H, H), jnp.bfloat16)],
        ),
        out_shape=jax.ShapeDtypeStruct((B, S, H), jnp.float32),
        compiler_params=pltpu.CompilerParams(
            vmem_limit_bytes=100 * 1024 * 1024,
        ),
    )(idx, w, x, expert_W, expert_W, bc)

    return y
